# baseline (speedup 1.0000x reference)
import functools

import jax
import jax.numpy as jnp
from jax import lax
from jax.experimental import pallas as pl
from jax.experimental.pallas import tpu as pltpu
from jax.experimental.pallas import tpu_sc as plsc

_NW = 32
_NC = 2


def _expand_tc(w, BLK=8000):
    V, D = w.shape

    def body(x_ref, o_ref):
        x = x_ref[...]
        o_ref[:, :D] = x
        o_ref[:, D:] = x

    return pl.pallas_call(
        body,
        grid=(V // BLK,),
        in_specs=[pl.BlockSpec((BLK, D), lambda i: (i, 0))],
        out_specs=pl.BlockSpec((BLK, 2 * D), lambda i: (i, 0)),
        out_shape=jax.ShapeDtypeStruct((V, 2 * D), jnp.float32),
    )(w)


def _reflow_tc(flat, B, H, D, SB=64):
    def body(x_ref, o_ref):
        x = x_ref[:, :D]
        o_ref[...] = x.reshape(SB, H, D)

    return pl.pallas_call(
        body,
        grid=(B // SB,),
        in_specs=[pl.BlockSpec((SB * H, 2 * D), lambda i: (i, 0))],
        out_specs=pl.BlockSpec((SB, H, D), lambda i: (i, 0, 0)),
        out_shape=jax.ShapeDtypeStruct((B, H, D), jnp.float32),
    )(flat)


def _pipeline(n_ch, nbuf, issue, wait, drain):
    n_main = (n_ch - nbuf) // nbuf

    for b in range(nbuf):
        issue(b, b)

    def block(jb, carry):
        jo = jb * nbuf
        for b in range(nbuf):
            wait(b)
            drain(jo + b, b)
            issue(jo + b + nbuf, b)
        return carry

    lax.fori_loop(0, n_main, block, 0)

    for j in range(n_main * nbuf, n_ch):
        b = j % nbuf
        wait(b)
        drain(j, b)
        if j + nbuf < n_ch:
            issue(j + nbuf, b)


def _make_gather_sc(N, V, W, NBUF):
    n_rows = N // W
    per_w = n_rows // _NW
    mesh = plsc.VectorSubcoreMesh(core_axis_name="c", subcore_axis_name="s")

    @functools.partial(
        pl.kernel,
        mesh=mesh,
        compiler_params=pltpu.CompilerParams(use_tc_tiling_on_sc=True),
        out_type=jax.ShapeDtypeStruct((N, W), jnp.float32),
        scratch_types=[
            pltpu.VMEM((per_w, W), jnp.int32),
            pltpu.VMEM((NBUF, W, W), jnp.float32),
            pltpu.SemaphoreType.DMA,
        ],
    )
    def k(idx_hbm, wide_hbm, out_hbm, idx_v, rows_v, sem):
        wid = lax.axis_index("s") * _NC + lax.axis_index("c")
        base = wid * per_w
        pltpu.sync_copy(idx_hbm.at[pl.ds(base, per_w)], idx_v)

        def issue(j, b):
            pltpu.async_copy(wide_hbm.at[idx_v.at[j]], rows_v.at[b], sem)

        def wait(b):
            pltpu.make_async_copy(
                wide_hbm.at[idx_v.at[0]], rows_v.at[b], sem
            ).wait()

        def drain(j, b):
            pltpu.sync_copy(
                rows_v.at[b], out_hbm.at[pl.ds((base + j) * W, W)]
            )

        _pipeline(per_w, NBUF, issue, wait, drain)

    return k


def kernel(input, weight_matrix):
    B, H = input.shape
    V, D = weight_matrix.shape
    N = B * H
    wide = _expand_tc(weight_matrix)
    idx2 = input.reshape(N // 128, 128).astype(jnp.int32)
    flat = _make_gather_sc(N, V, 128, NBUF=4)(idx2, wide)
    return _reflow_tc(flat, B, H, D)


# speedup vs baseline: 1.5269x; 1.5269x over previous
import functools

import jax
import jax.numpy as jnp
from jax import lax
from jax.experimental import pallas as pl
from jax.experimental.pallas import tpu as pltpu
from jax.experimental.pallas import tpu_sc as plsc

_NW = 32
_NC = 2


def _expand_tc(wt, BLK=4096):
    D, V = wt.shape

    def body(x_ref, o_ref):
        t = x_ref[...].T
        o_ref[:, :D] = t
        o_ref[:, D:] = t

    return pl.pallas_call(
        body,
        grid=(pl.cdiv(V, BLK),),
        in_specs=[pl.BlockSpec((D, BLK), lambda i: (0, i))],
        out_specs=pl.BlockSpec((BLK, 2 * D), lambda i: (i, 0)),
        out_shape=jax.ShapeDtypeStruct((V, 2 * D), jnp.float32),
    )(wt)


def _reflow_tc(flat, B, H, D, BB=2048):
    nb = B // BB

    def body(x_ref, o_ref):
        x = x_ref[:, :D]
        o_ref[...] = x.T.reshape(1, D, BB)

    return pl.pallas_call(
        body,
        grid=(H, nb),
        in_specs=[pl.BlockSpec((BB, 2 * D), lambda h, j: (h * nb + j, 0))],
        out_specs=pl.BlockSpec((1, D, BB), lambda h, j: (h, 0, j)),
        out_shape=jax.ShapeDtypeStruct((H, D, B), jnp.float32),
    )(flat)


def _pipeline(n_ch, nbuf, issue, wait, drain):
    n_main = (n_ch - nbuf) // nbuf

    for b in range(nbuf):
        issue(b, b)

    def block(jb, carry):
        jo = jb * nbuf
        for b in range(nbuf):
            wait(b)
            drain(jo + b, b)
            issue(jo + b + nbuf, b)
        return carry

    lax.fori_loop(0, n_main, block, 0)

    for j in range(n_main * nbuf, n_ch):
        b = j % nbuf
        wait(b)
        drain(j, b)
        if j + nbuf < n_ch:
            issue(j + nbuf, b)


def _make_gather_sc(N, V, W, NBUF):
    n_rows = N // W
    per_w = n_rows // _NW
    mesh = plsc.VectorSubcoreMesh(core_axis_name="c", subcore_axis_name="s")

    @functools.partial(
        pl.kernel,
        mesh=mesh,
        compiler_params=pltpu.CompilerParams(use_tc_tiling_on_sc=True),
        out_type=jax.ShapeDtypeStruct((N, W), jnp.float32),
        scratch_types=[
            pltpu.VMEM((per_w, W), jnp.int32),
            pltpu.VMEM((NBUF, W, W), jnp.float32),
            pltpu.SemaphoreType.DMA,
        ],
    )
    def k(idx_hbm, wide_hbm, out_hbm, idx_v, rows_v, sem):
        wid = lax.axis_index("s") * _NC + lax.axis_index("c")
        base = wid * per_w
        pltpu.sync_copy(idx_hbm.at[pl.ds(base, per_w)], idx_v)

        def issue(j, b):
            pltpu.async_copy(wide_hbm.at[idx_v.at[j]], rows_v.at[b], sem)

        def wait(b):
            pltpu.make_async_copy(
                wide_hbm.at[idx_v.at[0]], rows_v.at[b], sem
            ).wait()

        def drain(j, b):
            pltpu.sync_copy(
                rows_v.at[b], out_hbm.at[pl.ds((base + j) * W, W)]
            )

        _pipeline(per_w, NBUF, issue, wait, drain)

    return k


def kernel(input, weight_matrix):
    B, H = input.shape
    V, D = weight_matrix.shape
    N = B * H
    wide = _expand_tc(weight_matrix.T)
    idx2 = input.T.reshape(N // 128, 128).astype(jnp.int32)
    flat = _make_gather_sc(N, V, 128, NBUF=4)(idx2, wide)
    y = _reflow_tc(flat, B, H, D)
    return y.transpose(2, 0, 1)
